# Initial kernel scaffold; baseline (speedup 1.0000x reference)
import jax
import jax.numpy as jnp
from jax.experimental import pallas as pl


def kernel(x, edge_index, W1, b1, W2, b2, W3, b3):
    raise NotImplementedError("write your pallas kernel here")


# speedup vs baseline: 14.9588x; 14.9588x over previous
import functools

import jax
import jax.numpy as jnp
from jax import lax
from jax.experimental import pallas as pl
from jax.experimental.pallas import tpu as pltpu
from jax.experimental.pallas import tpu_sc as plsc

N = 10000
E = 320000
D = 128
NC, NS, L = 2, 16, 16
NW = NC * NS
KC = 128
CPT = 80
EPT = KC * CPT
E_PAD = NW * EPT
NROW = 10240
RPT = NROW // NS
MB = 1024
GRID = NROW // MB


def _mesh():
  return plsc.VectorSubcoreMesh(
      core_axis_name="c", subcore_axis_name="s",
      num_cores=NC, num_subcores=NS)


def _sc_degree(dst3):
  @functools.partial(
      pl.kernel,
      out_type=jax.ShapeDtypeStruct((NC, NROW), jnp.float32),
      mesh=_mesh(),
      scratch_types=[
          pltpu.VMEM((CPT, KC), jnp.int32),
          pltpu.VMEM((KC,), jnp.float32),
          pltpu.VMEM((RPT,), jnp.float32),
          pltpu.VMEM_SHARED((NROW,), jnp.float32),
      ],
  )
  def k(dst_hbm, deg_hbm, dst_v, ones_v, z_v, deg_sp):
    cid = lax.axis_index("c")
    sid = lax.axis_index("s")
    wid = sid * NC + cid

    def fill_z(i, c):
      z_v[pl.ds(i * L, L)] = jnp.zeros((L,), jnp.float32)
      return c
    lax.fori_loop(0, RPT // L, fill_z, 0)

    def fill_o(i, c):
      ones_v[pl.ds(i * L, L)] = jnp.ones((L,), jnp.float32)
      return c
    lax.fori_loop(0, KC // L, fill_o, 0)

    pltpu.sync_copy(z_v, deg_sp.at[pl.ds(sid * RPT, RPT)])
    plsc.subcore_barrier()

    pltpu.sync_copy(dst_hbm.at[wid], dst_v)

    def step(j, c):
      pltpu.sync_copy(ones_v, deg_sp.at[dst_v.at[j]], add=True)
      return c
    lax.fori_loop(0, CPT, step, 0)

    plsc.subcore_barrier()
    pltpu.sync_copy(deg_sp.at[pl.ds(sid * RPT, RPT)],
                    deg_hbm.at[cid, pl.ds(sid * RPT, RPT)])

  return k(dst3)


def _sc_gather_scatter(src3, dst3, xs):
  @functools.partial(
      pl.kernel,
      out_type=jax.ShapeDtypeStruct((NC, NROW, D), jnp.float32),
      mesh=_mesh(),
      scratch_types=[
          pltpu.VMEM((CPT, KC), jnp.int32),
          pltpu.VMEM((CPT, KC), jnp.int32),
          pltpu.VMEM((KC, D), jnp.float32),
          pltpu.VMEM_SHARED((NROW, D), jnp.float32),
      ],
  )
  def k(src_hbm, dst_hbm, xs_hbm, out_hbm, sidx, didx, rb, acc_sp):
    cid = lax.axis_index("c")
    sid = lax.axis_index("s")
    wid = sid * NC + cid

    def zrow(r, c):
      def zcol(i, c2):
        rb[r, pl.ds(i * L, L)] = jnp.zeros((L,), jnp.float32)
        return c2
      lax.fori_loop(0, D // L, zcol, 0)
      return c
    lax.fori_loop(0, KC, zrow, 0)

    def zacc(t, c):
      pltpu.sync_copy(rb, acc_sp.at[pl.ds(sid * RPT + t * KC, KC)])
      return c
    lax.fori_loop(0, RPT // KC, zacc, 0)
    plsc.subcore_barrier()

    pltpu.sync_copy(src_hbm.at[wid], sidx)
    pltpu.sync_copy(dst_hbm.at[wid], didx)

    def step(j, c):
      pltpu.sync_copy(xs_hbm.at[sidx.at[j]], rb)
      pltpu.sync_copy(rb, acc_sp.at[didx.at[j]], add=True)
      return c
    lax.fori_loop(0, CPT, step, 0)

    plsc.subcore_barrier()

    def wout(t, c):
      sl = pl.ds(sid * RPT + t * KC, KC)
      pltpu.sync_copy(acc_sp.at[sl], out_hbm.at[cid, sl])
      return c
    lax.fori_loop(0, RPT // KC, wout, 0)

  return k(src3, dst3, xs)


def _tc_scale(x, W1, degp):
  def body(x_ref, w_ref, deg_ref, xs_ref, dinv_ref):
    deg = deg_ref[0, :] + deg_ref[1, :] + 1.0
    dinv = lax.rsqrt(deg)
    xw = jnp.dot(x_ref[...], w_ref[...],
                 preferred_element_type=jnp.float32,
                 precision=lax.Precision.HIGHEST)
    xs_ref[...] = xw * dinv[:, None]
    dinv_ref[...] = dinv

  return pl.pallas_call(
      body,
      grid=(GRID,),
      in_specs=[
          pl.BlockSpec((MB, D), lambda i: (i, 0)),
          pl.BlockSpec((D, D), lambda i: (0, 0)),
          pl.BlockSpec((NC, MB), lambda i: (0, i)),
      ],
      out_specs=[
          pl.BlockSpec((MB, D), lambda i: (i, 0)),
          pl.BlockSpec((MB,), lambda i: (i,)),
      ],
      out_shape=[
          jax.ShapeDtypeStruct((N, D), jnp.float32),
          jax.ShapeDtypeStruct((N,), jnp.float32),
      ],
  )(x, W1, degp)


def _tc_mlp(acc, xs, dinv, b1, W2, b2, w3, b3):
  def body(acc_ref, xs_ref, dinv_ref, b1_ref, w2_ref, b2_ref, w3_ref,
           b3_ref, y_ref):
    a = acc_ref[0] + acc_ref[1] + xs_ref[...]
    dinv = dinv_ref[...]
    h = jnp.maximum(a * dinv[:, None] + b1_ref[...][None, :], 0.0)
    h = jnp.dot(h, w2_ref[...], preferred_element_type=jnp.float32,
                precision=lax.Precision.HIGHEST) + b2_ref[...][None, :]
    h = jnp.maximum(h, 0.0)
    y_ref[...] = jnp.sum(h * w3_ref[...][None, :], axis=1) + b3_ref[0]

  return pl.pallas_call(
      body,
      grid=(GRID,),
      in_specs=[
          pl.BlockSpec((NC, MB, D), lambda i: (0, i, 0)),
          pl.BlockSpec((MB, D), lambda i: (i, 0)),
          pl.BlockSpec((MB,), lambda i: (i,)),
          pl.BlockSpec((D,), lambda i: (0,)),
          pl.BlockSpec((D, D), lambda i: (0, 0)),
          pl.BlockSpec((D,), lambda i: (0,)),
          pl.BlockSpec((D,), lambda i: (0,)),
          pl.BlockSpec((1,), lambda i: (0,)),
      ],
      out_specs=pl.BlockSpec((MB,), lambda i: (i,)),
      out_shape=jax.ShapeDtypeStruct((N,), jnp.float32),
  )(acc, xs, dinv, b1, W2, b2, w3, b3)


def kernel(x, edge_index, W1, b1, W2, b2, W3, b3):
  src = edge_index[0].astype(jnp.int32)
  dst = edge_index[1].astype(jnp.int32)
  pad = E_PAD - E
  src_p = jnp.concatenate([src, jnp.zeros((pad,), jnp.int32)])
  dst_p = jnp.concatenate([dst, jnp.full((pad,), N, jnp.int32)])
  src3 = src_p.reshape(NW, CPT, KC)
  dst3 = dst_p.reshape(NW, CPT, KC)

  degp = _sc_degree(dst3)
  xs, dinv = _tc_scale(x, W1, degp)
  acc = _sc_gather_scatter(src3, dst3, xs)
  return _tc_mlp(acc, xs, dinv, b1, W2, b2, W3[:, 0], b3)
